# speedup vs baseline: 22.9970x; 22.9970x over previous
import functools

import jax
import jax.numpy as jnp
from jax import lax
from jax.experimental import pallas as pl
from jax.experimental.pallas import tpu as pltpu
from jax.experimental.pallas import tpu_sc as plsc

N = 10000
NPAD = 10240
E = 320000
NG = 16
NCORES = 2
NSUB = 16
WIN = 128
EPW = E // (NCORES * NSUB)
NFULL = EPW // WIN
TAIL = EPW - NFULL * WIN
STRIPE = NPAD // NSUB

_MESH = plsc.VectorSubcoreMesh(core_axis_name="c", subcore_axis_name="s")


def _fill2d(ref, rows, cols, value):
    vec = jnp.full((16,), value, jnp.float32)

    @pl.loop(0, rows)
    def _(r):
        @pl.loop(0, cols // 16)
        def _(cc):
            ref[r, pl.ds(cc * 16, 16)] = vec


def _zero_acc(acc_sh, zbuf, sid):
    d = zbuf.shape[1]
    _fill2d(zbuf, WIN, d, 0.0)

    @pl.loop(0, STRIPE // WIN)
    def _(j):
        pltpu.sync_copy(zbuf, acc_sh.at[pl.ds(sid * STRIPE + j * WIN, WIN)])


def _make_sc_deg():
    @functools.partial(
        pl.kernel,
        out_type=jax.ShapeDtypeStruct((NCORES, NPAD, 16), jnp.float32),
        mesh=_MESH,
        scratch_types=[
            pltpu.VMEM_SHARED((NPAD, 16), jnp.float32),
            pltpu.VMEM((WIN, 16), jnp.float32),
            pltpu.VMEM((WIN, 16), jnp.float32),
            pltpu.VMEM((TAIL, 16), jnp.float32),
            pltpu.VMEM((WIN,), jnp.int32),
            pltpu.VMEM((TAIL,), jnp.int32),
        ],
    )
    def sc_deg(dst_hbm, out_hbm, acc_sh, zbuf, ones_v, ones_t, idx_v, idx_t):
        cid = lax.axis_index("c")
        sid = lax.axis_index("s")
        _zero_acc(acc_sh, zbuf, sid)
        _fill2d(ones_v, WIN, 16, 1.0)
        _fill2d(ones_t, TAIL, 16, 1.0)
        plsc.subcore_barrier()

        base = cid * (NSUB * EPW) + sid * EPW

        @pl.loop(0, NFULL)
        def _(w):
            pltpu.sync_copy(dst_hbm.at[pl.ds(base + w * WIN, WIN)], idx_v)
            pltpu.sync_copy(ones_v, acc_sh.at[idx_v], add=True)

        pltpu.sync_copy(dst_hbm.at[pl.ds(base + NFULL * WIN, TAIL)], idx_t)
        pltpu.sync_copy(ones_t, acc_sh.at[idx_t], add=True)

        plsc.subcore_barrier()
        pltpu.sync_copy(
            acc_sh.at[pl.ds(sid * STRIPE, STRIPE)],
            out_hbm.at[cid, pl.ds(sid * STRIPE, STRIPE)],
        )

    return sc_deg


def _make_sc_papply(d):
    @functools.partial(
        pl.kernel,
        out_type=jax.ShapeDtypeStruct((NCORES, NPAD, d), jnp.float32),
        mesh=_MESH,
        scratch_types=[
            pltpu.VMEM_SHARED((NPAD, d), jnp.float32),
            pltpu.VMEM_SHARED((NPAD, d), jnp.float32),
            pltpu.VMEM((WIN, d), jnp.float32),
            pltpu.VMEM((TAIL, d), jnp.float32),
            pltpu.VMEM((WIN,), jnp.int32),
            pltpu.VMEM((WIN,), jnp.int32),
            pltpu.VMEM((TAIL,), jnp.int32),
            pltpu.VMEM((TAIL,), jnp.int32),
        ],
    )
    def sc_papply(tab_hbm, src_hbm, dst_hbm, out_hbm, tab_sh, acc_sh,
                  rows_v, rows_t, isrc_v, idst_v, isrc_t, idst_t):
        cid = lax.axis_index("c")
        sid = lax.axis_index("s")
        _zero_acc(acc_sh, rows_v, sid)
        pltpu.sync_copy(
            tab_hbm.at[pl.ds(sid * STRIPE, STRIPE)],
            tab_sh.at[pl.ds(sid * STRIPE, STRIPE)],
        )
        plsc.subcore_barrier()

        base = cid * (NSUB * EPW) + sid * EPW

        @pl.loop(0, NFULL)
        def _(w):
            pltpu.sync_copy(src_hbm.at[pl.ds(base + w * WIN, WIN)], isrc_v)
            pltpu.sync_copy(dst_hbm.at[pl.ds(base + w * WIN, WIN)], idst_v)
            pltpu.sync_copy(tab_sh.at[isrc_v], rows_v)
            pltpu.sync_copy(rows_v, acc_sh.at[idst_v], add=True)

        pltpu.sync_copy(src_hbm.at[pl.ds(base + NFULL * WIN, TAIL)], isrc_t)
        pltpu.sync_copy(dst_hbm.at[pl.ds(base + NFULL * WIN, TAIL)], idst_t)
        pltpu.sync_copy(tab_sh.at[isrc_t], rows_t)
        pltpu.sync_copy(rows_t, acc_sh.at[idst_t], add=True)

        plsc.subcore_barrier()
        pltpu.sync_copy(
            acc_sh.at[pl.ds(sid * STRIPE, STRIPE)],
            out_hbm.at[cid, pl.ds(sid * STRIPE, STRIPE)],
        )

    return sc_papply


_sc_deg = _make_sc_deg()
_sc_papply32 = _make_sc_papply(32)
_sc_papply16 = _make_sc_papply(16)


def _tc1_body(degp, x, w, dinv_o, hp_o):
    deg = degp[0, :, 0:1] + degp[1, :, 0:1] + 1.0
    dinv = lax.rsqrt(deg)
    dinv_o[...] = dinv
    h = jnp.dot(x[...], w[...], preferred_element_type=jnp.float32)
    hp_o[0:N, :] = dinv[0:N] * h
    hp_o[N:NPAD, :] = jnp.zeros((NPAD - N, 32), jnp.float32)


def _tc1(degp, x, w):
    return pl.pallas_call(
        _tc1_body,
        out_shape=(
            jax.ShapeDtypeStruct((NPAD, 1), jnp.float32),
            jax.ShapeDtypeStruct((NPAD, 32), jnp.float32),
        ),
    )(degp, x, w)


def _tc_mid_body(act, do, accp, hp, dinv, w, b, out_o):
    s = dinv[...] * (accp[0] + accp[1] + hp[...]) + b[...]
    a = act(s)
    h = jnp.dot(a, w[...], preferred_element_type=jnp.float32)
    out_o[0:N, :] = dinv[0:N] * h[0:N]
    out_o[N:NPAD, :] = jnp.zeros((NPAD - N, do), jnp.float32)


def _tc_mid(act, accp, hp, dinv, w, b):
    do = w.shape[1]
    return pl.pallas_call(
        functools.partial(_tc_mid_body, act, do),
        out_shape=jax.ShapeDtypeStruct((NPAD, do), jnp.float32),
    )(accp, hp, dinv, w, b)


def _tc_final_body(accp, hp, dinv, b4, batch, out_o):
    g = dinv[...] * (accp[0] + accp[1] + hp[...])
    g = g[0:N, 0:1] + b4[...]
    iot = lax.broadcasted_iota(jnp.int32, (N, NG), 1)
    m = iot == batch[...]
    sums = jnp.sum(jnp.where(m, g, 0.0), axis=0)
    cnts = jnp.sum(jnp.where(m, 1.0, 0.0), axis=0)
    out_o[0, :] = sums / jnp.maximum(cnts, 1.0)


def _tc_final(accp, hp, dinv, b4, batch):
    return pl.pallas_call(
        _tc_final_body,
        out_shape=jax.ShapeDtypeStruct((1, NG), jnp.float32),
    )(accp, hp, dinv, b4, batch)


def _pad2(w, r, c):
    return jnp.zeros((r, c), jnp.float32).at[: w.shape[0], : w.shape[1]].set(w)


def kernel(x, edge_index, edge_attr, batch, W1, b1, W2, b2, W3, b3, W4, b4):
    src = edge_index[0]
    dst = edge_index[1]

    W1p = _pad2(W1, 128, 32)
    W2p = _pad2(W2, 32, 32)
    W3p = _pad2(W3, 32, 16)
    W4p = _pad2(W4, 16, 16)
    b1p = _pad2(b1[None, :], 1, 32)
    b2p = _pad2(b2[None, :], 1, 32)
    b3p = _pad2(b3[None, :], 1, 16)
    b4p = b4[None, :]

    degp = _sc_deg(dst)
    dinv, h1p = _tc1(degp, x, W1p)
    acc1 = _sc_papply32(h1p, src, dst)
    h2p = _tc_mid(jax.nn.gelu, acc1, h1p, dinv, W2p, b1p)
    acc2 = _sc_papply32(h2p, src, dst)
    h3p = _tc_mid(jax.nn.relu, acc2, h2p, dinv, W3p, b2p)
    acc3 = _sc_papply16(h3p, src, dst)
    h4p = _tc_mid(jax.nn.gelu, acc3, h3p, dinv, W4p, b3p)
    acc4 = _sc_papply16(h4p, src, dst)
    out = _tc_final(acc4, h4p, dinv, b4p, batch.reshape(N, 1))
    return out.reshape(NG)


# speedup vs baseline: 44.9922x; 1.9564x over previous
import functools

import jax
import jax.numpy as jnp
from jax import lax
from jax.experimental import pallas as pl
from jax.experimental.pallas import tpu as pltpu
from jax.experimental.pallas import tpu_sc as plsc

N = 10000
NPAD = 10240
E = 320000
NG = 16
NCORES = 2
NSUB = 16
WIN = 128
NWIN = 80
EPAD = NCORES * NSUB * NWIN * WIN
K = 4
STRIPE = NPAD // NSUB

_MESH = plsc.VectorSubcoreMesh(core_axis_name="c", subcore_axis_name="s")


def _fill2d(ref, rows, cols, value):
    vec = jnp.full((16,), value, jnp.float32)

    @pl.loop(0, rows)
    def _(r):
        @pl.loop(0, cols // 16)
        def _(cc):
            ref[r, pl.ds(cc * 16, 16)] = vec


def _zero_acc(acc_sh, zbuf, sid):
    d = zbuf.shape[1]
    _fill2d(zbuf, WIN, d, 0.0)

    @pl.loop(0, STRIPE // WIN)
    def _(j):
        pltpu.sync_copy(zbuf, acc_sh.at[pl.ds(sid * STRIPE + j * WIN, WIN)])


def _make_sc_deg():
    @functools.partial(
        pl.kernel,
        out_type=jax.ShapeDtypeStruct((NCORES, NPAD, 16), jnp.float32),
        mesh=_MESH,
        scratch_types=[
            pltpu.VMEM_SHARED((NPAD, 16), jnp.float32),
            pltpu.VMEM((WIN, 16), jnp.float32),
            pltpu.VMEM((NWIN, WIN), jnp.int32),
            pltpu.SemaphoreType.DMA,
        ],
    )
    def sc_deg(dst_hbm, out_hbm, acc_sh, buf, idst, sem):
        cid = lax.axis_index("c")
        sid = lax.axis_index("s")
        wid = cid * NSUB + sid
        _zero_acc(acc_sh, buf, sid)
        _fill2d(buf, WIN, 16, 1.0)
        pltpu.sync_copy(dst_hbm.at[pl.ds(wid * NWIN, NWIN)], idst)
        plsc.subcore_barrier()

        @pl.loop(0, NWIN, step=8)
        def _(g):
            for b in range(8):
                pltpu.async_copy(buf, acc_sh.at[idst.at[g + b]], sem, add=True)
            for b in range(8):
                pltpu.make_async_copy(buf, acc_sh.at[idst.at[0]], sem).wait()

        plsc.subcore_barrier()
        pltpu.sync_copy(
            acc_sh.at[pl.ds(sid * STRIPE, STRIPE)],
            out_hbm.at[cid, pl.ds(sid * STRIPE, STRIPE)],
        )

    return sc_deg


def _make_sc_papply(d):
    @functools.partial(
        pl.kernel,
        out_type=jax.ShapeDtypeStruct((NCORES, NPAD, d), jnp.float32),
        mesh=_MESH,
        scratch_types=[
            pltpu.VMEM_SHARED((NPAD, d), jnp.float32),
            pltpu.VMEM_SHARED((NPAD, d), jnp.float32),
            pltpu.VMEM((K, WIN, d), jnp.float32),
            pltpu.VMEM((NWIN, WIN), jnp.int32),
            pltpu.VMEM((NWIN, WIN), jnp.int32),
            pltpu.SemaphoreType.DMA((K,)),
            pltpu.SemaphoreType.DMA((K,)),
        ],
    )
    def sc_papply(tab_hbm, src_hbm, dst_hbm, out_hbm, tab_sh, acc_sh,
                  rows, isrc, idst, gsem, ssem):
        cid = lax.axis_index("c")
        sid = lax.axis_index("s")
        wid = cid * NSUB + sid
        _zero_acc(acc_sh, rows.at[0], sid)
        pltpu.sync_copy(
            tab_hbm.at[pl.ds(sid * STRIPE, STRIPE)],
            tab_sh.at[pl.ds(sid * STRIPE, STRIPE)],
        )
        pltpu.sync_copy(src_hbm.at[pl.ds(wid * NWIN, NWIN)], isrc)
        pltpu.sync_copy(dst_hbm.at[pl.ds(wid * NWIN, NWIN)], idst)
        plsc.subcore_barrier()

        def gather_start(j, b):
            pltpu.async_copy(tab_sh.at[isrc.at[j]], rows.at[b], gsem.at[b])

        def gather_wait(b):
            pltpu.make_async_copy(
                tab_sh.at[isrc.at[0]], rows.at[b], gsem.at[b]).wait()

        def scat_start(j, b):
            pltpu.async_copy(rows.at[b], acc_sh.at[idst.at[j]], ssem.at[b],
                             add=True)

        def scat_wait(b):
            pltpu.make_async_copy(
                rows.at[b], acc_sh.at[idst.at[0]], ssem.at[b]).wait()

        for b in range(K):
            gather_start(b, b)

        @pl.loop(0, NWIN - K, step=K)
        def _(g):
            for b in range(K):
                gather_wait(b)
                scat_start(g + b, b)
            for b in range(K):
                scat_wait(b)
                gather_start(g + K + b, b)

        for b in range(K):
            gather_wait(b)
            scat_start(NWIN - K + b, b)
        for b in range(K):
            scat_wait(b)

        plsc.subcore_barrier()
        pltpu.sync_copy(
            acc_sh.at[pl.ds(sid * STRIPE, STRIPE)],
            out_hbm.at[cid, pl.ds(sid * STRIPE, STRIPE)],
        )

    return sc_papply


_sc_deg = _make_sc_deg()
_sc_papply32 = _make_sc_papply(32)
_sc_papply16 = _make_sc_papply(16)


def _tc1_body(degp, x, w, dinv_o, hp_o):
    deg = degp[0, :, 0:1] + degp[1, :, 0:1] + 1.0
    dinv = lax.rsqrt(deg)
    dinv_o[...] = dinv
    h = jnp.dot(x[...], w[...], preferred_element_type=jnp.float32)
    hp_o[0:N, :] = dinv[0:N] * h
    hp_o[N:NPAD, :] = jnp.zeros((NPAD - N, 32), jnp.float32)


def _tc1(degp, x, w):
    return pl.pallas_call(
        _tc1_body,
        out_shape=(
            jax.ShapeDtypeStruct((NPAD, 1), jnp.float32),
            jax.ShapeDtypeStruct((NPAD, 32), jnp.float32),
        ),
    )(degp, x, w)


def _tc_mid_body(act, do, accp, hp, dinv, w, b, out_o):
    s = dinv[...] * (accp[0] + accp[1] + hp[...]) + b[...]
    a = act(s)
    h = jnp.dot(a, w[...], preferred_element_type=jnp.float32)
    out_o[0:N, :] = dinv[0:N] * h[0:N]
    out_o[N:NPAD, :] = jnp.zeros((NPAD - N, do), jnp.float32)


def _tc_mid(act, accp, hp, dinv, w, b):
    do = w.shape[1]
    return pl.pallas_call(
        functools.partial(_tc_mid_body, act, do),
        out_shape=jax.ShapeDtypeStruct((NPAD, do), jnp.float32),
    )(accp, hp, dinv, w, b)


def _tc_final_body(accp, hp, dinv, b4, batch, out_o):
    g = dinv[...] * (accp[0] + accp[1] + hp[...])
    g = g[0:N, 0:1] + b4[...]
    iot = lax.broadcasted_iota(jnp.int32, (N, NG), 1)
    m = iot == batch[...]
    sums = jnp.sum(jnp.where(m, g, 0.0), axis=0)
    cnts = jnp.sum(jnp.where(m, 1.0, 0.0), axis=0)
    out_o[0, :] = sums / jnp.maximum(cnts, 1.0)


def _tc_final(accp, hp, dinv, b4, batch):
    return pl.pallas_call(
        _tc_final_body,
        out_shape=jax.ShapeDtypeStruct((1, NG), jnp.float32),
    )(accp, hp, dinv, b4, batch)


def _pad2(w, r, c):
    return jnp.zeros((r, c), jnp.float32).at[: w.shape[0], : w.shape[1]].set(w)


def kernel(x, edge_index, edge_attr, batch, W1, b1, W2, b2, W3, b3, W4, b4):
    pad = jnp.full((EPAD - E,), N, jnp.int32)
    src = jnp.concatenate([edge_index[0], pad]).reshape(EPAD // WIN, WIN)
    dst = jnp.concatenate([edge_index[1], pad]).reshape(EPAD // WIN, WIN)

    W1p = _pad2(W1, 128, 32)
    W2p = _pad2(W2, 32, 32)
    W3p = _pad2(W3, 32, 16)
    W4p = _pad2(W4, 16, 16)
    b1p = _pad2(b1[None, :], 1, 32)
    b2p = _pad2(b2[None, :], 1, 32)
    b3p = _pad2(b3[None, :], 1, 16)
    b4p = b4[None, :]

    degp = _sc_deg(dst)
    dinv, h1p = _tc1(degp, x, W1p)
    acc1 = _sc_papply32(h1p, src, dst)
    h2p = _tc_mid(jax.nn.gelu, acc1, h1p, dinv, W2p, b1p)
    acc2 = _sc_papply32(h2p, src, dst)
    h3p = _tc_mid(jax.nn.relu, acc2, h2p, dinv, W3p, b2p)
    acc3 = _sc_papply16(h3p, src, dst)
    h4p = _tc_mid(jax.nn.gelu, acc3, h3p, dinv, W4p, b3p)
    acc4 = _sc_papply16(h4p, src, dst)
    out = _tc_final(acc4, h4p, dinv, b4p, batch.reshape(N, 1))
    return out.reshape(NG)


# speedup vs baseline: 45.1530x; 1.0036x over previous
import functools

import jax
import jax.numpy as jnp
from jax import lax
from jax.experimental import pallas as pl
from jax.experimental.pallas import tpu as pltpu
from jax.experimental.pallas import tpu_sc as plsc

N = 10000
NPAD = 10240
E = 320000
NG = 16
NCORES = 2
NSUB = 16
WIN = 128
NWIN = 80
EPAD = NCORES * NSUB * NWIN * WIN
K = 4
STRIPE = NPAD // NSUB

BLK = 1280
NB = NPAD // BLK

_MESH = plsc.VectorSubcoreMesh(core_axis_name="c", subcore_axis_name="s")


def _fill2d(ref, rows, cols, value):
    vec = jnp.full((16,), value, jnp.float32)

    @pl.loop(0, rows)
    def _(r):
        @pl.loop(0, cols // 16)
        def _(cc):
            ref[r, pl.ds(cc * 16, 16)] = vec


def _zero_acc(acc_sh, zbuf, sid):
    d = zbuf.shape[1]
    _fill2d(zbuf, WIN, d, 0.0)

    @pl.loop(0, STRIPE // WIN)
    def _(j):
        pltpu.sync_copy(zbuf, acc_sh.at[pl.ds(sid * STRIPE + j * WIN, WIN)])


def _make_sc_deg():
    @functools.partial(
        pl.kernel,
        out_type=jax.ShapeDtypeStruct((NCORES, NPAD, 16), jnp.float32),
        mesh=_MESH,
        scratch_types=[
            pltpu.VMEM_SHARED((NPAD, 16), jnp.float32),
            pltpu.VMEM((WIN, 16), jnp.float32),
            pltpu.VMEM((NWIN, WIN), jnp.int32),
            pltpu.SemaphoreType.DMA,
        ],
    )
    def sc_deg(dst_hbm, out_hbm, acc_sh, buf, idst, sem):
        cid = lax.axis_index("c")
        sid = lax.axis_index("s")
        wid = cid * NSUB + sid
        _zero_acc(acc_sh, buf, sid)
        _fill2d(buf, WIN, 16, 1.0)
        pltpu.sync_copy(dst_hbm.at[pl.ds(wid * NWIN, NWIN)], idst)
        plsc.subcore_barrier()

        @pl.loop(0, NWIN, step=8)
        def _(g):
            for b in range(8):
                pltpu.async_copy(buf, acc_sh.at[idst.at[g + b]], sem, add=True)
            for b in range(8):
                pltpu.make_async_copy(buf, acc_sh.at[idst.at[0]], sem).wait()

        plsc.subcore_barrier()
        pltpu.sync_copy(
            acc_sh.at[pl.ds(sid * STRIPE, STRIPE)],
            out_hbm.at[cid, pl.ds(sid * STRIPE, STRIPE)],
        )

    return sc_deg


def _make_sc_papply(d):
    @functools.partial(
        pl.kernel,
        out_type=jax.ShapeDtypeStruct((NCORES, NPAD, d), jnp.float32),
        mesh=_MESH,
        scratch_types=[
            pltpu.VMEM_SHARED((NPAD, d), jnp.float32),
            pltpu.VMEM_SHARED((NPAD, d), jnp.float32),
            pltpu.VMEM((K, WIN, d), jnp.float32),
            pltpu.VMEM((NWIN, WIN), jnp.int32),
            pltpu.VMEM((NWIN, WIN), jnp.int32),
            pltpu.SemaphoreType.DMA((K,)),
            pltpu.SemaphoreType.DMA((K,)),
        ],
    )
    def sc_papply(tab_hbm, src_hbm, dst_hbm, out_hbm, tab_sh, acc_sh,
                  rows, isrc, idst, gsem, ssem):
        cid = lax.axis_index("c")
        sid = lax.axis_index("s")
        wid = cid * NSUB + sid
        _zero_acc(acc_sh, rows.at[0], sid)
        pltpu.sync_copy(
            tab_hbm.at[pl.ds(sid * STRIPE, STRIPE)],
            tab_sh.at[pl.ds(sid * STRIPE, STRIPE)],
        )
        pltpu.sync_copy(src_hbm.at[pl.ds(wid * NWIN, NWIN)], isrc)
        pltpu.sync_copy(dst_hbm.at[pl.ds(wid * NWIN, NWIN)], idst)
        plsc.subcore_barrier()

        def gather_start(j, b):
            pltpu.async_copy(tab_sh.at[isrc.at[j]], rows.at[b], gsem.at[b])

        def gather_wait(b):
            pltpu.make_async_copy(
                tab_sh.at[isrc.at[0]], rows.at[b], gsem.at[b]).wait()

        def scat_start(j, b):
            pltpu.async_copy(rows.at[b], acc_sh.at[idst.at[j]], ssem.at[b],
                             add=True)

        def scat_wait(b):
            pltpu.make_async_copy(
                rows.at[b], acc_sh.at[idst.at[0]], ssem.at[b]).wait()

        for b in range(K):
            gather_start(b, b)

        @pl.loop(0, NWIN - K, step=K)
        def _(g):
            for b in range(K):
                gather_wait(b)
                scat_start(g + b, b)
            for b in range(K):
                scat_wait(b)
                gather_start(g + K + b, b)

        for b in range(K):
            gather_wait(b)
            scat_start(NWIN - K + b, b)
        for b in range(K):
            scat_wait(b)

        plsc.subcore_barrier()
        pltpu.sync_copy(
            acc_sh.at[pl.ds(sid * STRIPE, STRIPE)],
            out_hbm.at[cid, pl.ds(sid * STRIPE, STRIPE)],
        )

    return sc_papply


_sc_deg = _make_sc_deg()
_sc_papply32 = _make_sc_papply(32)
_sc_papply16 = _make_sc_papply(16)


def _tc1_body(degp, x, w, dinv_o, hp_o):
    deg = degp[0, :, 0:1] + degp[1, :, 0:1] + 1.0
    dinv = lax.rsqrt(deg)
    dinv_o[...] = dinv
    h = jnp.dot(x[...], w[...], preferred_element_type=jnp.float32)
    hp_o[...] = dinv * h


def _tc1(degp, x, w):
    do = w.shape[1]
    return pl.pallas_call(
        _tc1_body,
        grid=(NB,),
        in_specs=[
            pl.BlockSpec((2, BLK, 16), lambda i: (0, i, 0)),
            pl.BlockSpec((BLK, 128), lambda i: (i, 0)),
            pl.BlockSpec((128, do), lambda i: (0, 0)),
        ],
        out_specs=(
            pl.BlockSpec((BLK, 1), lambda i: (i, 0)),
            pl.BlockSpec((BLK, do), lambda i: (i, 0)),
        ),
        out_shape=(
            jax.ShapeDtypeStruct((NPAD, 1), jnp.float32),
            jax.ShapeDtypeStruct((NPAD, do), jnp.float32),
        ),
    )(degp, x, w)


def _tc_mid_body(act, accp, hp, dinv, w, b, out_o):
    s = dinv[...] * (accp[0] + accp[1] + hp[...]) + b[...]
    a = act(s)
    h = jnp.dot(a, w[...], preferred_element_type=jnp.float32)
    out_o[...] = dinv[...] * h


def _tc_mid(act, accp, hp, dinv, w, b):
    di, do = w.shape
    return pl.pallas_call(
        functools.partial(_tc_mid_body, act),
        grid=(NB,),
        in_specs=[
            pl.BlockSpec((2, BLK, di), lambda i: (0, i, 0)),
            pl.BlockSpec((BLK, di), lambda i: (i, 0)),
            pl.BlockSpec((BLK, 1), lambda i: (i, 0)),
            pl.BlockSpec((di, do), lambda i: (0, 0)),
            pl.BlockSpec((1, di), lambda i: (0, 0)),
        ],
        out_specs=pl.BlockSpec((BLK, do), lambda i: (i, 0)),
        out_shape=jax.ShapeDtypeStruct((NPAD, do), jnp.float32),
    )(accp, hp, dinv, w, b)


def _tc_final_body(accp, hp, dinv, b4, batch, out_o, s_acc, c_acc):
    i = pl.program_id(0)

    @pl.when(i == 0)
    def _():
        s_acc[...] = jnp.zeros_like(s_acc)
        c_acc[...] = jnp.zeros_like(c_acc)

    g = dinv[...] * (accp[0] + accp[1] + hp[...])
    g = g[:, 0:1] + b4[...]
    iot = lax.broadcasted_iota(jnp.int32, (BLK, NG), 1)
    m = iot == batch[...]
    s_acc[0, :] += jnp.sum(jnp.where(m, g, 0.0), axis=0)
    c_acc[0, :] += jnp.sum(jnp.where(m, 1.0, 0.0), axis=0)

    @pl.when(i == NB - 1)
    def _():
        out_o[0, :] = s_acc[0, :] / jnp.maximum(c_acc[0, :], 1.0)


def _tc_final(accp, hp, dinv, b4, batch):
    return pl.pallas_call(
        _tc_final_body,
        grid=(NB,),
        in_specs=[
            pl.BlockSpec((2, BLK, 16), lambda i: (0, i, 0)),
            pl.BlockSpec((BLK, 16), lambda i: (i, 0)),
            pl.BlockSpec((BLK, 1), lambda i: (i, 0)),
            pl.BlockSpec((1, 1), lambda i: (0, 0)),
            pl.BlockSpec((BLK, 1), lambda i: (i, 0)),
        ],
        out_specs=pl.BlockSpec((1, NG), lambda i: (0, 0)),
        out_shape=jax.ShapeDtypeStruct((1, NG), jnp.float32),
        scratch_shapes=[
            pltpu.VMEM((1, NG), jnp.float32),
            pltpu.VMEM((1, NG), jnp.float32),
        ],
    )(accp, hp, dinv, b4, batch)


def _pad2(w, r, c):
    return jnp.zeros((r, c), jnp.float32).at[: w.shape[0], : w.shape[1]].set(w)


def kernel(x, edge_index, edge_attr, batch, W1, b1, W2, b2, W3, b3, W4, b4):
    pad = jnp.full((EPAD - E,), N, jnp.int32)
    src = jnp.concatenate([edge_index[0], pad]).reshape(EPAD // WIN, WIN)
    dst = jnp.concatenate([edge_index[1], pad]).reshape(EPAD // WIN, WIN)
    xp = jnp.concatenate([x, jnp.zeros((NPAD - N, 128), jnp.float32)])
    batchp = jnp.concatenate(
        [batch, jnp.full((NPAD - N,), NG, jnp.int32)]).reshape(NPAD, 1)

    W1p = _pad2(W1, 128, 32)
    W2p = _pad2(W2, 32, 32)
    W3p = _pad2(W3, 32, 16)
    W4p = _pad2(W4, 16, 16)
    b1p = _pad2(b1[None, :], 1, 32)
    b2p = _pad2(b2[None, :], 1, 32)
    b3p = _pad2(b3[None, :], 1, 16)
    b4p = b4[None, :]

    degp = _sc_deg(dst)
    dinv, h1p = _tc1(degp, xp, W1p)
    acc1 = _sc_papply32(h1p, src, dst)
    h2p = _tc_mid(jax.nn.gelu, acc1, h1p, dinv, W2p, b1p)
    acc2 = _sc_papply32(h2p, src, dst)
    h3p = _tc_mid(jax.nn.relu, acc2, h2p, dinv, W3p, b2p)
    acc3 = _sc_papply16(h3p, src, dst)
    h4p = _tc_mid(jax.nn.gelu, acc3, h3p, dinv, W4p, b3p)
    acc4 = _sc_papply16(h4p, src, dst)
    out = _tc_final(acc4, h4p, dinv, b4p, batchp)
    return out.reshape(NG)
